# Initial kernel scaffold; baseline (speedup 1.0000x reference)
import jax
import jax.numpy as jnp
from jax.experimental import pallas as pl


def kernel(x, edge_index, batch, W1, b1, W2, b2, W3, b3, Wfc, bfc):
    raise NotImplementedError("write your pallas kernel here")


# speedup vs baseline: 5.9357x; 5.9357x over previous
import functools

import jax
import jax.numpy as jnp
from jax import lax
from jax.experimental import pallas as pl
from jax.experimental.pallas import tpu as pltpu
from jax.experimental.pallas import tpu_sc as plsc

N = 10000
E = 320000
DIN = 128
DH = 256
DOUT = 128
G = 64

NP = 10240
NC = 2
NS = 16
CHUNK = 128
GRP = 16
CPS = 160
KT = NS * CPS
E_PAD = KT * CHUNK
ZPS = NP // NS
TRASH = N
HALF = DH // NC

DEG_W = NC * NS
DEG_EDGES = E_PAD // DEG_W

BLK = 2048
NBLK = NP // BLK

_mesh = plsc.VectorSubcoreMesh(core_axis_name="c", subcore_axis_name="s")


@functools.partial(
    pl.kernel,
    out_type=jax.ShapeDtypeStruct((DEG_W, 1, NP), jnp.float32),
    mesh=_mesh,
    scratch_types=[
        pltpu.VMEM((DEG_EDGES,), jnp.int32),
        pltpu.VMEM((NP,), jnp.float32),
    ],
    compiler_params=pltpu.CompilerParams(needs_layout_passes=False),
)
def _deg_kernel(dst_hbm, out_hbm, dvm, acc):
    c = lax.axis_index("c")
    s = lax.axis_index("s")
    w = s * NC + c

    zero16 = jnp.zeros((16,), jnp.float32)

    def zbody(i, carry):
        acc[pl.ds(i * 16, 16)] = zero16
        return carry

    lax.fori_loop(0, NP // 16, zbody, 0)

    pltpu.sync_copy(dst_hbm.at[pl.ds(w * DEG_EDGES, DEG_EDGES)], dvm)

    one16 = jnp.ones((16,), jnp.float32)

    def body(i, carry):
        idx = dvm[pl.ds(i * 16, 16)]
        plsc.addupdate_scatter(acc, [idx], one16)
        return carry

    lax.fori_loop(0, DEG_EDGES // 16, body, 0)

    pltpu.sync_copy(acc, out_hbm.at[w, 0])


@functools.partial(
    pl.kernel,
    out_type=jax.ShapeDtypeStruct((NC, NP, HALF), jnp.float32),
    mesh=_mesh,
    scratch_types=[
        pltpu.VMEM_SHARED((NP, HALF), jnp.float32),
        pltpu.VMEM((GRP, CHUNK), jnp.int32),
        pltpu.VMEM((GRP, CHUNK), jnp.int32),
        pltpu.VMEM((CHUNK, HALF), jnp.float32),
        pltpu.SemaphoreType.DMA,
    ],
)
def _spmm_kernel(src_hbm, dst_hbm, tbl_hbm, zeros_hbm, out_hbm,
                 acc, srcv, dstv, rows, sem):
    c = lax.axis_index("c")
    s = lax.axis_index("s")

    pltpu.sync_copy(zeros_hbm, acc.at[pl.ds(s * ZPS, ZPS)])
    plsc.subcore_barrier()

    def grp(g, carry):
        base = s * CPS + g * GRP
        pltpu.sync_copy(src_hbm.at[c, pl.ds(base, GRP)], srcv)
        pltpu.sync_copy(dst_hbm.at[pl.ds(base, GRP)], dstv)

        def body(j, carry2):
            pltpu.async_copy(tbl_hbm.at[srcv.at[j]], rows, sem).wait()
            pltpu.sync_copy(rows, acc.at[dstv.at[j]], add=True)
            return carry2

        lax.fori_loop(0, GRP, body, 0)
        return carry

    lax.fori_loop(0, CPS // GRP, grp, 0)

    plsc.subcore_barrier()
    pltpu.sync_copy(acc.at[pl.ds(s * ZPS, ZPS)],
                    out_hbm.at[c, pl.ds(s * ZPS, ZPS)])


def _tc1_body(x_ref, w_ref, part_ref, hw_ref, hws_ref, dinv_ref):
    i = pl.program_id(0)
    deg = 1.0 + jnp.sum(part_ref[:, pl.ds(i * BLK, BLK)], axis=0)
    dinv = lax.rsqrt(deg)
    dinv_ref[...] = dinv[:, None]
    hw = jnp.dot(x_ref[...], w_ref[...], preferred_element_type=jnp.float32)
    hw_ref[0] = hw[:, :HALF]
    hw_ref[1] = hw[:, HALF:]
    hws_ref[0] = hw[:, :HALF] * dinv[:, None]
    hws_ref[1] = hw[:, HALF:] * dinv[:, None]


_tc1 = pl.pallas_call(
    _tc1_body,
    grid=(NBLK,),
    in_specs=[
        pl.BlockSpec((BLK, DIN), lambda i: (i, 0)),
        pl.BlockSpec((DIN, DH), lambda i: (0, 0)),
        pl.BlockSpec((DEG_W, NP), lambda i: (0, 0)),
    ],
    out_specs=[
        pl.BlockSpec((NC, BLK, HALF), lambda i: (0, i, 0)),
        pl.BlockSpec((NC, BLK, HALF), lambda i: (0, i, 0)),
        pl.BlockSpec((BLK, 1), lambda i: (i, 0)),
    ],
    out_shape=[
        jax.ShapeDtypeStruct((NC, NP, HALF), jnp.float32),
        jax.ShapeDtypeStruct((NC, NP, HALF), jnp.float32),
        jax.ShapeDtypeStruct((NP, 1), jnp.float32),
    ],
)


def _tc2_body(es_ref, hwp_ref, dinv_ref, b_ref, w_ref, hwn_ref, hwsn_ref):
    dinv = dinv_ref[...]
    es = jnp.concatenate([es_ref[0], es_ref[1]], axis=1)
    hwp = jnp.concatenate([hwp_ref[0], hwp_ref[1]], axis=1)
    h = jnp.maximum(dinv * es + (dinv * dinv) * hwp + b_ref[...], 0.0)
    hwn = jnp.dot(h, w_ref[...], preferred_element_type=jnp.float32)
    hwn_ref[0] = hwn[:, :HALF]
    hwn_ref[1] = hwn[:, HALF:]
    hwsn_ref[0] = hwn[:, :HALF] * dinv
    hwsn_ref[1] = hwn[:, HALF:] * dinv


_tc2 = pl.pallas_call(
    _tc2_body,
    grid=(NBLK,),
    in_specs=[
        pl.BlockSpec((NC, BLK, HALF), lambda i: (0, i, 0)),
        pl.BlockSpec((NC, BLK, HALF), lambda i: (0, i, 0)),
        pl.BlockSpec((BLK, 1), lambda i: (i, 0)),
        pl.BlockSpec((1, DH), lambda i: (0, 0)),
        pl.BlockSpec((DH, DH), lambda i: (0, 0)),
    ],
    out_specs=[
        pl.BlockSpec((NC, BLK, HALF), lambda i: (0, i, 0)),
        pl.BlockSpec((NC, BLK, HALF), lambda i: (0, i, 0)),
    ],
    out_shape=[
        jax.ShapeDtypeStruct((NC, NP, HALF), jnp.float32),
        jax.ShapeDtypeStruct((NC, NP, HALF), jnp.float32),
    ],
)


def _tc3_body(es_ref, hwp_ref, dinv_ref, b_ref, batch_ref, wfc_ref, bfc_ref,
              out_ref, sums, cnts):
    i = pl.program_id(0)

    @pl.when(i == 0)
    def _():
        sums[...] = jnp.zeros_like(sums)
        cnts[...] = jnp.zeros_like(cnts)

    dinv = dinv_ref[...]
    es = jnp.concatenate([es_ref[0], es_ref[1]], axis=1)
    hwp = jnp.concatenate([hwp_ref[0], hwp_ref[1]], axis=1)
    h = jnp.maximum(dinv * es + (dinv * dinv) * hwp + b_ref[...], 0.0)
    bb = batch_ref[0, 0, :]
    p = (lax.broadcasted_iota(jnp.int32, (G, BLK), 0) == bb[None, :]
         ).astype(jnp.float32)
    sums[...] += jnp.dot(p, h, preferred_element_type=jnp.float32)
    cnts[...] += jnp.broadcast_to(jnp.sum(p, axis=1, keepdims=True), (G, HALF))

    @pl.when(i == NBLK - 1)
    def _():
        pooled = sums[...] / jnp.maximum(cnts[...][:, :1], 1.0)
        out_ref[...] = (jnp.dot(pooled, wfc_ref[...],
                                preferred_element_type=jnp.float32)
                        + bfc_ref[...])


_tc3 = pl.pallas_call(
    _tc3_body,
    grid=(NBLK,),
    in_specs=[
        pl.BlockSpec((NC, BLK, HALF), lambda i: (0, i, 0)),
        pl.BlockSpec((NC, BLK, HALF), lambda i: (0, i, 0)),
        pl.BlockSpec((BLK, 1), lambda i: (i, 0)),
        pl.BlockSpec((1, DH), lambda i: (0, 0)),
        pl.BlockSpec((1, 1, BLK), lambda i: (i, 0, 0)),
        pl.BlockSpec((DH, DOUT), lambda i: (0, 0)),
        pl.BlockSpec((1, DOUT), lambda i: (0, 0)),
    ],
    out_specs=pl.BlockSpec((G, DOUT), lambda i: (0, 0)),
    out_shape=jax.ShapeDtypeStruct((G, DOUT), jnp.float32),
    scratch_shapes=[
        pltpu.VMEM((G, DH), jnp.float32),
        pltpu.VMEM((G, HALF), jnp.float32),
    ],
)


def kernel(x, edge_index, batch, W1, b1, W2, b2, W3, b3, Wfc, bfc):
    src = edge_index[0].astype(jnp.int32)
    dst = edge_index[1].astype(jnp.int32)
    padlen = E_PAD - E
    src_p = jnp.concatenate([src, jnp.zeros((padlen,), jnp.int32)])
    dst_p = jnp.concatenate([dst, jnp.full((padlen,), TRASH, jnp.int32)])
    src_g = jnp.stack([src_p, src_p + NP]).reshape(NC, KT, CHUNK)
    dst_g = dst_p.reshape(KT, CHUNK)
    zeros_h = jnp.zeros((ZPS, HALF), jnp.float32)

    x_p = jnp.concatenate([x, jnp.zeros((NP - N, DIN), jnp.float32)])
    batch3 = jnp.concatenate(
        [batch.astype(jnp.int32), jnp.full((NP - N,), G, jnp.int32)]
    ).reshape(NBLK, 1, BLK)

    part = _deg_kernel(dst_p).reshape(DEG_W, NP)

    b1r = b1.reshape(1, DH)
    b2r = b2.reshape(1, DH)
    b3r = b3.reshape(1, DH)
    bfcr = bfc.reshape(1, DOUT)

    hw1, hws1, dinv = _tc1(x_p, W1, part)
    es1 = _spmm_kernel(src_g, dst_g, hws1.reshape(NC * NP, HALF), zeros_h)
    hw2, hws2 = _tc2(es1, hw1, dinv, b1r, W2)
    es2 = _spmm_kernel(src_g, dst_g, hws2.reshape(NC * NP, HALF), zeros_h)
    hw3, hws3 = _tc2(es2, hw2, dinv, b2r, W3)
    es3 = _spmm_kernel(src_g, dst_g, hws3.reshape(NC * NP, HALF), zeros_h)
    return _tc3(es3, hw3, dinv, b3r, batch3, Wfc, bfcr)


# speedup vs baseline: 6.3301x; 1.0664x over previous
import functools

import jax
import jax.numpy as jnp
from jax import lax
from jax.experimental import pallas as pl
from jax.experimental.pallas import tpu as pltpu
from jax.experimental.pallas import tpu_sc as plsc

N = 10000
E = 320000
DIN = 128
DH = 256
DOUT = 128
G = 64

NP = 10240
NC = 2
NS = 16
CHUNK = 128
GRP = 16
CPS = 160
KT = NS * CPS
E_PAD = KT * CHUNK
ZPS = NP // NS
TRASH = N
HALF = DH // NC

DEG_W = NC * NS
DEG_EDGES = E_PAD // DEG_W

BLK = 2048
NBLK = NP // BLK

_mesh = plsc.VectorSubcoreMesh(core_axis_name="c", subcore_axis_name="s")


@functools.partial(
    pl.kernel,
    out_type=jax.ShapeDtypeStruct((DEG_W, 1, NP), jnp.float32),
    mesh=_mesh,
    scratch_types=[
        pltpu.VMEM((DEG_EDGES,), jnp.int32),
        pltpu.VMEM((NP,), jnp.float32),
    ],
    compiler_params=pltpu.CompilerParams(needs_layout_passes=False),
)
def _deg_kernel(dst_hbm, out_hbm, dvm, acc):
    c = lax.axis_index("c")
    s = lax.axis_index("s")
    w = s * NC + c

    zero16 = jnp.zeros((16,), jnp.float32)

    def zbody(i, carry):
        acc[pl.ds(i * 16, 16)] = zero16
        return carry

    lax.fori_loop(0, NP // 16, zbody, 0)

    pltpu.sync_copy(dst_hbm.at[pl.ds(w * DEG_EDGES, DEG_EDGES)], dvm)

    one16 = jnp.ones((16,), jnp.float32)

    def body(i, carry):
        idx = dvm[pl.ds(i * 16, 16)]
        plsc.addupdate_scatter(acc, [idx], one16)
        return carry

    lax.fori_loop(0, DEG_EDGES // 16, body, 0)

    pltpu.sync_copy(acc, out_hbm.at[w, 0])


@functools.partial(
    pl.kernel,
    out_type=jax.ShapeDtypeStruct((NC, NP, HALF), jnp.float32),
    mesh=_mesh,
    scratch_types=[
        pltpu.VMEM_SHARED((NP, HALF), jnp.float32),
        pltpu.VMEM((GRP, CHUNK), jnp.int32),
        pltpu.VMEM((GRP, CHUNK), jnp.int32),
        pltpu.VMEM((CHUNK, HALF), jnp.float32),
        pltpu.VMEM((CHUNK, HALF), jnp.float32),
        pltpu.SemaphoreType.DMA,
        pltpu.SemaphoreType.DMA,
        pltpu.SemaphoreType.DMA,
        pltpu.SemaphoreType.DMA,
    ],
)
def _spmm_kernel(src_hbm, dst_hbm, tbl_hbm, zeros_hbm, out_hbm,
                 acc, srcv, dstv, rows_a, rows_b, sga, sgb, ssa, ssb):
    c = lax.axis_index("c")
    s = lax.axis_index("s")

    pltpu.sync_copy(zeros_hbm, acc.at[pl.ds(s * ZPS, ZPS)])
    plsc.subcore_barrier()

    def grp(g, carry):
        base = s * CPS + g * GRP
        pltpu.sync_copy(src_hbm.at[c, pl.ds(base, GRP)], srcv)
        pltpu.sync_copy(dst_hbm.at[pl.ds(base, GRP)], dstv)

        def pair(p, carry2):
            j0 = p * 2
            j1 = p * 2 + 1
            first = jnp.logical_and(g == 0, p == 0)

            @pl.when(jnp.logical_not(first))
            def _():
                pltpu.make_async_copy(rows_a, acc.at[dstv.at[j0]], ssa).wait()
                pltpu.make_async_copy(rows_b, acc.at[dstv.at[j1]], ssb).wait()

            ga = pltpu.async_copy(tbl_hbm.at[srcv.at[j0]], rows_a, sga)
            gb = pltpu.async_copy(tbl_hbm.at[srcv.at[j1]], rows_b, sgb)
            ga.wait()
            pltpu.async_copy(rows_a, acc.at[dstv.at[j0]], ssa, add=True)
            gb.wait()
            pltpu.async_copy(rows_b, acc.at[dstv.at[j1]], ssb, add=True)
            return carry2

        lax.fori_loop(0, GRP // 2, pair, 0)
        return carry

    lax.fori_loop(0, CPS // GRP, grp, 0)
    pltpu.make_async_copy(rows_a, acc.at[dstv.at[GRP - 2]], ssa).wait()
    pltpu.make_async_copy(rows_b, acc.at[dstv.at[GRP - 1]], ssb).wait()

    plsc.subcore_barrier()
    pltpu.sync_copy(acc.at[pl.ds(s * ZPS, ZPS)],
                    out_hbm.at[c, pl.ds(s * ZPS, ZPS)])


def _tc1_body(x_ref, w_ref, part_ref, hw_ref, hws_ref, dinv_ref):
    i = pl.program_id(0)
    deg = 1.0 + jnp.sum(part_ref[:, pl.ds(i * BLK, BLK)], axis=0)
    dinv = lax.rsqrt(deg)
    dinv_ref[...] = dinv[:, None]
    hw = jnp.dot(x_ref[...], w_ref[...], preferred_element_type=jnp.float32)
    hw_ref[0] = hw[:, :HALF]
    hw_ref[1] = hw[:, HALF:]
    hws_ref[0] = hw[:, :HALF] * dinv[:, None]
    hws_ref[1] = hw[:, HALF:] * dinv[:, None]


_tc1 = pl.pallas_call(
    _tc1_body,
    grid=(NBLK,),
    in_specs=[
        pl.BlockSpec((BLK, DIN), lambda i: (i, 0)),
        pl.BlockSpec((DIN, DH), lambda i: (0, 0)),
        pl.BlockSpec((DEG_W, NP), lambda i: (0, 0)),
    ],
    out_specs=[
        pl.BlockSpec((NC, BLK, HALF), lambda i: (0, i, 0)),
        pl.BlockSpec((NC, BLK, HALF), lambda i: (0, i, 0)),
        pl.BlockSpec((BLK, 1), lambda i: (i, 0)),
    ],
    out_shape=[
        jax.ShapeDtypeStruct((NC, NP, HALF), jnp.float32),
        jax.ShapeDtypeStruct((NC, NP, HALF), jnp.float32),
        jax.ShapeDtypeStruct((NP, 1), jnp.float32),
    ],
)


def _tc2_body(es_ref, hwp_ref, dinv_ref, b_ref, w_ref, hwn_ref, hwsn_ref):
    dinv = dinv_ref[...]
    es = jnp.concatenate([es_ref[0], es_ref[1]], axis=1)
    hwp = jnp.concatenate([hwp_ref[0], hwp_ref[1]], axis=1)
    h = jnp.maximum(dinv * es + (dinv * dinv) * hwp + b_ref[...], 0.0)
    hwn = jnp.dot(h, w_ref[...], preferred_element_type=jnp.float32)
    hwn_ref[0] = hwn[:, :HALF]
    hwn_ref[1] = hwn[:, HALF:]
    hwsn_ref[0] = hwn[:, :HALF] * dinv
    hwsn_ref[1] = hwn[:, HALF:] * dinv


_tc2 = pl.pallas_call(
    _tc2_body,
    grid=(NBLK,),
    in_specs=[
        pl.BlockSpec((NC, BLK, HALF), lambda i: (0, i, 0)),
        pl.BlockSpec((NC, BLK, HALF), lambda i: (0, i, 0)),
        pl.BlockSpec((BLK, 1), lambda i: (i, 0)),
        pl.BlockSpec((1, DH), lambda i: (0, 0)),
        pl.BlockSpec((DH, DH), lambda i: (0, 0)),
    ],
    out_specs=[
        pl.BlockSpec((NC, BLK, HALF), lambda i: (0, i, 0)),
        pl.BlockSpec((NC, BLK, HALF), lambda i: (0, i, 0)),
    ],
    out_shape=[
        jax.ShapeDtypeStruct((NC, NP, HALF), jnp.float32),
        jax.ShapeDtypeStruct((NC, NP, HALF), jnp.float32),
    ],
)


def _tc3_body(es_ref, hwp_ref, dinv_ref, b_ref, batch_ref, wfc_ref, bfc_ref,
              out_ref, sums, cnts):
    i = pl.program_id(0)

    @pl.when(i == 0)
    def _():
        sums[...] = jnp.zeros_like(sums)
        cnts[...] = jnp.zeros_like(cnts)

    dinv = dinv_ref[...]
    es = jnp.concatenate([es_ref[0], es_ref[1]], axis=1)
    hwp = jnp.concatenate([hwp_ref[0], hwp_ref[1]], axis=1)
    h = jnp.maximum(dinv * es + (dinv * dinv) * hwp + b_ref[...], 0.0)
    bb = batch_ref[0, 0, :]
    p = (lax.broadcasted_iota(jnp.int32, (G, BLK), 0) == bb[None, :]
         ).astype(jnp.float32)
    sums[...] += jnp.dot(p, h, preferred_element_type=jnp.float32)
    cnts[...] += jnp.broadcast_to(jnp.sum(p, axis=1, keepdims=True), (G, HALF))

    @pl.when(i == NBLK - 1)
    def _():
        pooled = sums[...] / jnp.maximum(cnts[...][:, :1], 1.0)
        out_ref[...] = (jnp.dot(pooled, wfc_ref[...],
                                preferred_element_type=jnp.float32)
                        + bfc_ref[...])


_tc3 = pl.pallas_call(
    _tc3_body,
    grid=(NBLK,),
    in_specs=[
        pl.BlockSpec((NC, BLK, HALF), lambda i: (0, i, 0)),
        pl.BlockSpec((NC, BLK, HALF), lambda i: (0, i, 0)),
        pl.BlockSpec((BLK, 1), lambda i: (i, 0)),
        pl.BlockSpec((1, DH), lambda i: (0, 0)),
        pl.BlockSpec((1, 1, BLK), lambda i: (i, 0, 0)),
        pl.BlockSpec((DH, DOUT), lambda i: (0, 0)),
        pl.BlockSpec((1, DOUT), lambda i: (0, 0)),
    ],
    out_specs=pl.BlockSpec((G, DOUT), lambda i: (0, 0)),
    out_shape=jax.ShapeDtypeStruct((G, DOUT), jnp.float32),
    scratch_shapes=[
        pltpu.VMEM((G, DH), jnp.float32),
        pltpu.VMEM((G, HALF), jnp.float32),
    ],
)


def kernel(x, edge_index, batch, W1, b1, W2, b2, W3, b3, Wfc, bfc):
    src = edge_index[0].astype(jnp.int32)
    dst = edge_index[1].astype(jnp.int32)
    padlen = E_PAD - E
    src_p = jnp.concatenate([src, jnp.zeros((padlen,), jnp.int32)])
    dst_p = jnp.concatenate([dst, jnp.full((padlen,), TRASH, jnp.int32)])
    src_g = jnp.stack([src_p, src_p + NP]).reshape(NC, KT, CHUNK)
    dst_g = dst_p.reshape(KT, CHUNK)
    zeros_h = jnp.zeros((ZPS, HALF), jnp.float32)

    x_p = jnp.concatenate([x, jnp.zeros((NP - N, DIN), jnp.float32)])
    batch3 = jnp.concatenate(
        [batch.astype(jnp.int32), jnp.full((NP - N,), G, jnp.int32)]
    ).reshape(NBLK, 1, BLK)

    part = _deg_kernel(dst_p).reshape(DEG_W, NP)

    b1r = b1.reshape(1, DH)
    b2r = b2.reshape(1, DH)
    b3r = b3.reshape(1, DH)
    bfcr = bfc.reshape(1, DOUT)

    hw1, hws1, dinv = _tc1(x_p, W1, part)
    es1 = _spmm_kernel(src_g, dst_g, hws1.reshape(NC * NP, HALF), zeros_h)
    hw2, hws2 = _tc2(es1, hw1, dinv, b1r, W2)
    es2 = _spmm_kernel(src_g, dst_g, hws2.reshape(NC * NP, HALF), zeros_h)
    hw3, hws3 = _tc2(es2, hw2, dinv, b2r, W3)
    es3 = _spmm_kernel(src_g, dst_g, hws3.reshape(NC * NP, HALF), zeros_h)
    return _tc3(es3, hw3, dinv, b3r, batch3, Wfc, bfcr)
